# baseline (speedup 1.0000x reference)
import functools

import jax
import jax.numpy as jnp
from jax.experimental import pallas as pl

_EMB = 1024
_E = 16
_BLK = 4096


def _gate_kernel(h_ref, u_ref, W_ref, b_ref, o_ref):
    h = h_ref[...]
    u = u_ref[...]
    Wh = W_ref[:_EMB, :]
    Wu = W_ref[_EMB:, :]
    g = (
        jax.lax.dot(h, Wh, preferred_element_type=jnp.float32)
        + jax.lax.dot(u, Wu, preferred_element_type=jnp.float32)
        + b_ref[...]
    )
    m = jnp.max(g, axis=-1, keepdims=True)
    iota = jax.lax.broadcasted_iota(jnp.int32, g.shape, 1)
    i1 = jnp.min(jnp.where(g == m, iota, _E), axis=-1, keepdims=True)
    e = jnp.exp(g - m)
    e_rest = jnp.where(iota == i1, -1.0, e)
    e2 = jnp.max(e_rest, axis=-1, keepdims=True)
    S = jnp.sum(e, axis=-1, keepdims=True)
    r = 1.0 / (1.0 + e2 + 1e-9 * S)
    keep = (iota == i1) | (e_rest >= e2)
    o_ref[...] = jnp.where(keep, e * r, 0.0)


@jax.jit
def kernel(h, u, W, b):
    n = h.shape[0]
    grid = (n // _BLK,)
    return pl.pallas_call(
        _gate_kernel,
        grid=grid,
        in_specs=[
            pl.BlockSpec((_BLK, _EMB), lambda i: (i, 0)),
            pl.BlockSpec((_BLK, u.shape[1]), lambda i: (i, 0)),
            pl.BlockSpec(W.shape, lambda i: (0, 0)),
            pl.BlockSpec(b.shape, lambda i: (0,)),
        ],
        out_specs=pl.BlockSpec((_BLK, _E), lambda i: (i, 0)),
        out_shape=jax.ShapeDtypeStruct((n, _E), jnp.float32),
    )(h, u, W, b)
